# baseline (speedup 1.0000x reference)
import functools

import jax
import jax.numpy as jnp
from jax import lax
from jax.experimental import pallas as pl
from jax.experimental.pallas import tpu as pltpu
from jax.experimental.pallas import tpu_sc as plsc

_NC, _NS = 2, 16
_NW = _NC * _NS
_NG = 8


def _tc_pre(x_nodes, u_pad, cvec, bv):
    Vp, C = x_nodes.shape
    K16 = u_pad.shape[1]

    def body(x_ref, u_ref, c_ref, a_ref, t_ref):
        x = x_ref[...]
        ux = jnp.dot(x, u_ref[...], preferred_element_type=jnp.float32)
        mask9 = (lax.broadcasted_iota(jnp.int32, (1, K16), 1) < 9).astype(
            jnp.float32)
        a_ref[...] = jnp.exp(ux + c_ref[...]) * mask9
        gv = jnp.exp(-ux) * mask9
        t_ref[...] = jnp.concatenate(
            [x, gv, jnp.zeros((bv, CW - C - K16), jnp.float32)], axis=1)

    CW = 2 * C
    return pl.pallas_call(
        body,
        grid=(Vp // bv,),
        in_specs=[
            pl.BlockSpec((bv, C), lambda i: (i, 0)),
            pl.BlockSpec((C, K16), lambda i: (0, 0)),
            pl.BlockSpec((1, K16), lambda i: (0, 0)),
        ],
        out_specs=[
            pl.BlockSpec((bv, K16), lambda i: (i, 0)),
            pl.BlockSpec((bv, CW), lambda i: (i, 0)),
        ],
        out_shape=[
            jax.ShapeDtypeStruct((Vp, K16), jnp.float32),
            jax.ShapeDtypeStruct((Vp, CW), jnp.float32),
        ],
    )(x_nodes, u_pad, cvec)


def _sc_attention(adj2, xcomb, a, C, K, NB):
    Vp = a.shape[0]
    CW = xcomb.shape[1]
    K16 = a.shape[1]
    vpw = Vp // _NW
    n_grp = vpw // _NG
    rows_g = _NG * NB
    YW = K * C
    mesh = plsc.VectorSubcoreMesh(core_axis_name="c", subcore_axis_name="s",
                                  num_cores=_NC, num_subcores=_NS)
    nC = C // 16
    nH = 2
    nCh = nC // nH

    @functools.partial(
        pl.kernel,
        out_type=jax.ShapeDtypeStruct((Vp, YW), jnp.float32),
        mesh=mesh,
        scratch_types=(
            [pltpu.VMEM((n_grp, rows_g), jnp.int32),
             pltpu.VMEM((2, _NG, K16), jnp.float32),
             pltpu.VMEM((2, rows_g, CW), jnp.float32),
             pltpu.VMEM((2, _NG, YW), jnp.float32),
             pltpu.VMEM((NB, K16), jnp.float32)]
            + [pltpu.SemaphoreType.DMA] * 4
        ),
    )
    def sc_kernel(adj_hbm, x_hbm, a_hbm, y_hbm,
                  ibuf, abuf, xg, ybuf, wbuf,
                  sx0, sx1, wy0, wy1):
        sx, wy = (sx0, sx1), (wy0, wy1)
        wid = lax.axis_index("s") * _NC + lax.axis_index("c")
        nbase = wid * vpw
        pltpu.sync_copy(adj_hbm.at[pl.ds(wid * n_grp, n_grp)], ibuf)

        def fire(g, b):
            pltpu.async_copy(x_hbm.at[ibuf.at[g]], xg.at[b], sx[b])
            pltpu.async_copy(a_hbm.at[pl.ds(nbase + g * _NG, _NG)],
                             abuf.at[b], sx[b])

        def do_group(g, b):
            pltpu.make_async_copy(x_hbm.at[ibuf.at[g]], xg.at[b],
                                  sx[b]).wait()
            pltpu.make_async_copy(a_hbm.at[pl.ds(nbase + g * _NG, _NG)],
                                  abuf.at[b], sx[b]).wait()

            @pl.when(g >= 2)
            def _():
                pltpu.make_async_copy(
                    ybuf.at[b], y_hbm.at[pl.ds(nbase + (g - 2) * _NG, _NG)],
                    wy[b]).wait()

            lane = lax.iota(jnp.int32, 16)

            @pl.loop(0, _NG)
            def _(ln):
                row0 = ln * NB
                av = abuf[b, ln, :]
                for n in range(NB):
                    ev = av * xg[b, row0 + n, pl.ds(C, 16)]
                    d = ev
                    for sh in (8, 4, 2, 1):
                        d = d + d.at[lane ^ sh].get(
                            mode="promise_in_bounds")
                    wbuf[n, :] = ev / d
                KR = 5
                for h in range(nH):
                    acc = [[None] * nCh for _ in range(KR)]
                    for n in range(NB):
                        wv = wbuf[n, :]
                        xr = [xg[b, row0 + n, pl.ds(h * nCh * 16 + r * 16, 16)]
                              for r in range(nCh)]
                        for k in range(K):
                            wk = wv[k]
                            for r in range(nCh):
                                t = wk * xr[r]
                                if k < KR:
                                    acc[k][r] = t if n == 0 else acc[k][r] + t
                                else:
                                    ds = pl.ds(k * C + h * nCh * 16 + r * 16,
                                               16)
                                    if n == 0:
                                        ybuf[b, ln, ds] = t
                                    else:
                                        plsc.addupdate(ybuf.at[b, ln, ds], t)
                    for k in range(KR):
                        for r in range(nCh):
                            ybuf[b, ln,
                                 pl.ds(k * C + h * nCh * 16 + r * 16, 16)] = (
                                acc[k][r])

            @pl.when(g + 2 < n_grp)
            def _():
                fire(g + 2, b)

            pltpu.async_copy(ybuf.at[b],
                             y_hbm.at[pl.ds(nbase + g * _NG, _NG)], wy[b])

        fire(0, 0)
        fire(1, 1)

        @pl.loop(0, n_grp, step=2)
        def _(t0):
            do_group(t0, 0)
            do_group(t0 + 1, 1)

        for b in range(2):
            pltpu.make_async_copy(
                ybuf.at[b],
                y_hbm.at[pl.ds(nbase + (n_grp - 2 + b) * _NG, _NG)],
                wy[b]).wait()

    return sc_kernel(adj2, xcomb, a)


def _tc_post(y, adj_p, wt9, b2, bv, V_out):
    Vp, YW = y.shape
    NB = adj_p.shape[1]
    O = wt9.shape[1]

    def body(y_ref, adj_ref, wt_ref, b_ref, o_ref):
        o1 = jnp.dot(y_ref[...], wt_ref[...],
                     preferred_element_type=jnp.float32)
        cnt = jnp.sum((adj_ref[...] != 0).astype(jnp.float32),
                      axis=1, keepdims=True)
        rcnt = jnp.where(cnt > 0.0, 1.0 / jnp.maximum(cnt, 1.0), 0.0)
        o_ref[...] = jnp.maximum(o1 * rcnt + b_ref[...], 0.0)

    return pl.pallas_call(
        body,
        grid=(Vp // bv,),
        in_specs=[
            pl.BlockSpec((bv, YW), lambda i: (i, 0)),
            pl.BlockSpec((bv, NB), lambda i: (i, 0)),
            pl.BlockSpec((YW, O), lambda i: (0, 0)),
            pl.BlockSpec((1, O), lambda i: (0, 0)),
        ],
        out_specs=pl.BlockSpec((bv, O), lambda i: (i, 0)),
        out_shape=jax.ShapeDtypeStruct((V_out, O), jnp.float32),
    )(y, adj_p, wt9, b2)


def kernel(inputs, adj, W, b, u, c):
    B, V, C = inputs.shape
    K = u.shape[1]
    O = W.shape[2]
    NB = adj.shape[1]
    K16 = 16
    bv = 512

    x = inputs.reshape(V, C)
    Vp = ((V + bv - 1) // bv) * bv
    x_nodes = jnp.pad(x, ((0, Vp - V), (0, 0)))
    adj_p = jnp.pad(adj, ((0, Vp - V), (0, 0)))
    idx = jnp.where(adj_p > 0, adj_p - 1, V).astype(jnp.int32)
    adj2 = idx.reshape(Vp * NB // (_NG * NB), _NG * NB)

    u_pad = jnp.pad(u, ((0, 0), (0, K16 - K)))
    cvec = jnp.pad(c, (0, K16 - K)).reshape(1, K16)
    a_tab, xcomb = _tc_pre(x_nodes, u_pad, cvec, bv)

    y = _sc_attention(adj2, xcomb, a_tab, C, K, NB)

    wt9 = jnp.transpose(W, (1, 0, 2)).reshape(K * C, O)
    b2 = b.reshape(1, O)
    out = _tc_post(y, adj_p, wt9, b2, bv, V)
    return out.reshape(B, V, O)


# speedup vs baseline: 1.0105x; 1.0105x over previous
import functools

import jax
import jax.numpy as jnp
from jax import lax
from jax.experimental import pallas as pl
from jax.experimental.pallas import tpu as pltpu
from jax.experimental.pallas import tpu_sc as plsc

_NC, _NS = 2, 16
_NW = _NC * _NS
_NG = 8


def _tc_pre(x_nodes, u_pad, cvec, bv):
    Vp, C = x_nodes.shape
    K16 = u_pad.shape[1]

    def body(x_ref, u_ref, c_ref, a_ref, g_ref):
        ux = jnp.dot(x_ref[...], u_ref[...], preferred_element_type=jnp.float32)
        mask9 = (lax.broadcasted_iota(jnp.int32, (1, K16), 1) < 9).astype(
            jnp.float32)
        a_ref[...] = jnp.exp(ux + c_ref[...]) * mask9
        gv = jnp.exp(-ux) * mask9
        g_ref[...] = jnp.pad(gv, ((0, 0), (0, K16)))

    return pl.pallas_call(
        body,
        grid=(Vp // bv,),
        in_specs=[
            pl.BlockSpec((bv, C), lambda i: (i, 0)),
            pl.BlockSpec((C, K16), lambda i: (0, 0)),
            pl.BlockSpec((1, K16), lambda i: (0, 0)),
        ],
        out_specs=[
            pl.BlockSpec((bv, K16), lambda i: (i, 0)),
            pl.BlockSpec((bv, 2 * K16), lambda i: (i, 0)),
        ],
        out_shape=[
            jax.ShapeDtypeStruct((Vp, K16), jnp.float32),
            jax.ShapeDtypeStruct((Vp, 2 * K16), jnp.float32),
        ],
    )(x_nodes, u_pad, cvec)


def _sc_attention(adj2, xcomb, a, C, K, NB):
    Vp = a.shape[0]
    CW = xcomb.shape[1]
    K16 = a.shape[1]
    vpw = Vp // _NW
    n_grp = vpw // _NG
    rows_g = _NG * NB
    YW = K * C
    mesh = plsc.VectorSubcoreMesh(core_axis_name="c", subcore_axis_name="s",
                                  num_cores=_NC, num_subcores=_NS)
    nC = C // 16
    nH = 2
    nCh = nC // nH

    @functools.partial(
        pl.kernel,
        out_type=jax.ShapeDtypeStruct((Vp, YW), jnp.float32),
        mesh=mesh,
        scratch_types=(
            [pltpu.VMEM((n_grp, rows_g), jnp.int32),
             pltpu.VMEM((2, _NG, K16), jnp.float32),
             pltpu.VMEM((2, rows_g, CW), jnp.float32),
             pltpu.VMEM((2, _NG, YW), jnp.float32),
             pltpu.VMEM((NB, K16), jnp.float32)]
            + [pltpu.SemaphoreType.DMA] * 4
        ),
    )
    def sc_kernel(adj_hbm, x_hbm, a_hbm, y_hbm,
                  ibuf, abuf, xg, ybuf, wbuf,
                  sx0, sx1, wy0, wy1):
        sx, wy = (sx0, sx1), (wy0, wy1)
        wid = lax.axis_index("s") * _NC + lax.axis_index("c")
        nbase = wid * vpw
        pltpu.sync_copy(adj_hbm.at[pl.ds(wid * n_grp, n_grp)], ibuf)

        def fire(g, b):
            pltpu.async_copy(x_hbm.at[ibuf.at[g]], xg.at[b], sx[b])
            pltpu.async_copy(a_hbm.at[pl.ds(nbase + g * _NG, _NG)],
                             abuf.at[b], sx[b])

        def do_group(g, b):
            pltpu.make_async_copy(x_hbm.at[ibuf.at[g]], xg.at[b],
                                  sx[b]).wait()
            pltpu.make_async_copy(a_hbm.at[pl.ds(nbase + g * _NG, _NG)],
                                  abuf.at[b], sx[b]).wait()

            @pl.when(g >= 2)
            def _():
                pltpu.make_async_copy(
                    ybuf.at[b], y_hbm.at[pl.ds(nbase + (g - 2) * _NG, _NG)],
                    wy[b]).wait()

            lane = lax.iota(jnp.int32, 16)

            @pl.loop(0, _NG)
            def _(ln):
                row0 = ln * NB
                av = abuf[b, ln, :]
                for n in range(NB):
                    ev = av * xg[b, row0 + n, pl.ds(C, 16)]
                    d = ev
                    for sh in (8, 4, 2, 1):
                        d = d + d.at[lane ^ sh].get(
                            mode="promise_in_bounds")
                    wbuf[n, :] = ev / d
                for h in range(nH):
                    acc = [[None] * nCh for _ in range(K)]
                    for n in range(NB):
                        wv = wbuf[n, :]
                        xr = [xg[b, row0 + n, pl.ds(h * nCh * 16 + r * 16, 16)]
                              for r in range(nCh)]
                        for k in range(K):
                            wk = wv[k]
                            for r in range(nCh):
                                t = wk * xr[r]
                                acc[k][r] = t if n == 0 else acc[k][r] + t
                    for k in range(K):
                        for r in range(nCh):
                            ybuf[b, ln,
                                 pl.ds(k * C + h * nCh * 16 + r * 16, 16)] = (
                                acc[k][r])

            @pl.when(g + 2 < n_grp)
            def _():
                fire(g + 2, b)

            pltpu.async_copy(ybuf.at[b],
                             y_hbm.at[pl.ds(nbase + g * _NG, _NG)], wy[b])

        fire(0, 0)
        fire(1, 1)

        @pl.loop(0, n_grp, step=2)
        def _(t0):
            do_group(t0, 0)
            do_group(t0 + 1, 1)

        for b in range(2):
            pltpu.make_async_copy(
                ybuf.at[b],
                y_hbm.at[pl.ds(nbase + (n_grp - 2 + b) * _NG, _NG)],
                wy[b]).wait()

    return sc_kernel(adj2, xcomb, a)


def _tc_post(y, adj_p, wt9, b2, bv):
    Vp, YW = y.shape
    NB = adj_p.shape[1]
    O = wt9.shape[1]

    def body(y_ref, adj_ref, wt_ref, b_ref, o_ref):
        o1 = jnp.dot(y_ref[...], wt_ref[...],
                     preferred_element_type=jnp.float32)
        cnt = jnp.sum((adj_ref[...] != 0).astype(jnp.float32),
                      axis=1, keepdims=True)
        rcnt = jnp.where(cnt > 0.0, 1.0 / jnp.maximum(cnt, 1.0), 0.0)
        o_ref[...] = jnp.maximum(o1 * rcnt + b_ref[...], 0.0)

    return pl.pallas_call(
        body,
        grid=(Vp // bv,),
        in_specs=[
            pl.BlockSpec((bv, YW), lambda i: (i, 0)),
            pl.BlockSpec((bv, NB), lambda i: (i, 0)),
            pl.BlockSpec((YW, O), lambda i: (0, 0)),
            pl.BlockSpec((1, O), lambda i: (0, 0)),
        ],
        out_specs=pl.BlockSpec((bv, O), lambda i: (i, 0)),
        out_shape=jax.ShapeDtypeStruct((Vp, O), jnp.float32),
    )(y, adj_p, wt9, b2)


def kernel(inputs, adj, W, b, u, c):
    B, V, C = inputs.shape
    K = u.shape[1]
    O = W.shape[2]
    NB = adj.shape[1]
    K16 = 16
    bv = 512

    x = inputs.reshape(V, C)
    Vp = ((V + bv - 1) // bv) * bv
    x_nodes = jnp.pad(x, ((0, Vp - V), (0, 0)))
    x_zpad = jnp.concatenate([jnp.zeros((1, C), jnp.float32), x], axis=0)
    adj_p = jnp.pad(adj, ((0, Vp - V), (0, 0)))
    adj2 = adj_p.reshape(Vp * NB // (_NG * NB), _NG * NB)

    u_pad = jnp.pad(u, ((0, 0), (0, K16 - K)))
    cvec = jnp.pad(c, (0, K16 - K)).reshape(1, K16)
    a_tab, g_body = _tc_pre(x_nodes, u_pad, cvec, bv)
    mask9_row = jnp.pad((jnp.arange(K16) < K).astype(jnp.float32),
                        (0, K16)).reshape(1, 2 * K16)
    g_pad = jnp.concatenate([mask9_row, g_body[:V]], axis=0)
    xcomb = jnp.concatenate(
        [x_zpad, g_pad, jnp.zeros((V + 1, C - 2 * K16), jnp.float32)], axis=1)

    y = _sc_attention(adj2, xcomb, a_tab, C, K, NB)

    wt9 = jnp.transpose(W, (1, 0, 2)).reshape(K * C, O)
    b2 = b.reshape(1, O)
    out = _tc_post(y, adj_p, wt9, b2, bv)
    return out[:V].reshape(B, V, O)
